# baseline (speedup 1.0000x reference)
import functools

import jax
import jax.numpy as jnp
from jax import lax
from jax.experimental import pallas as pl
from jax.experimental.pallas import tpu as pltpu
from jax.experimental.pallas import tpu_sc as plsc

NC, NS, L = 2, 16, 16
NW = NC * NS


@functools.lru_cache(maxsize=None)
def _make_kernel(B, N, G):
    P = B * N
    assert P % NW == 0
    PW = P // NW
    assert N % PW == 0
    K = 2000
    assert PW % K == 0 and K % L == 0 and K % 8 == 0
    NCHUNK = PW // K
    NG = K // L
    GG = G * G
    TILES_PER_BATCH = NW // B

    mesh = plsc.VectorSubcoreMesh(core_axis_name="c", subcore_axis_name="s")

    @functools.partial(
        pl.kernel,
        out_type=jax.ShapeDtypeStruct((P, 2), jnp.float32),
        mesh=mesh,
        compiler_params=pltpu.CompilerParams(use_tc_tiling_on_sc=False,
                                             needs_layout_passes=False),
        scratch_types=[
            pltpu.VMEM((K,), jnp.float32),
            pltpu.VMEM((K,), jnp.float32),
            pltpu.VMEM((4 * K,), jnp.int32),
            pltpu.VMEM((K,), jnp.float32),
            pltpu.VMEM((K,), jnp.float32),
            pltpu.VMEM((K,), jnp.float32),
            pltpu.VMEM((K,), jnp.float32),
            pltpu.VMEM((4 * K,), jnp.float32),
            pltpu.VMEM((4 * K,), jnp.float32),
            pltpu.VMEM((K, 2), jnp.float32),
            pltpu.SemaphoreType.DMA,
        ],
    )
    def grid_sample(t0, t1, u_hbm, v_hbm, out_hbm,
                    u_v, v_v, idx_v, wa_v, wb_v, wc_v, wd_v, g0_v, g1_v, o_v,
                    sem):
        cid = lax.axis_index("c")
        sid = lax.axis_index("s")
        wid = sid * NC + cid
        batch = wid // TILES_PER_BATCH
        boff = batch * GG
        lanes = lax.iota(jnp.int32, L)
        col0 = jnp.zeros((L,), jnp.int32)
        col1 = jnp.ones((L,), jnp.int32)

        def chunk_body(ci, carry):
            base = wid * PW + ci * K
            pltpu.sync_copy(u_hbm.at[pl.ds(base, K)], u_v)
            pltpu.sync_copy(v_hbm.at[pl.ds(base, K)], v_v)

            def build(g, c2):
                sl = pl.ds(g * L, L)
                u = u_v[sl]
                v = v_v[sl]
                ui = jnp.minimum(u.astype(jnp.int32), G - 2)
                vi = jnp.minimum(v.astype(jnp.int32), G - 2)
                fu = u - ui.astype(jnp.float32)
                fv = v - vi.astype(jnp.float32)
                gu = 1.0 - fu
                gv = 1.0 - fv
                ia = ui * G + vi + boff
                idx_v[sl] = ia
                idx_v[pl.ds(K + g * L, L)] = ia + 1
                idx_v[pl.ds(2 * K + g * L, L)] = ia + G
                idx_v[pl.ds(3 * K + g * L, L)] = ia + G + 1
                wa_v[sl] = gu * gv
                wb_v[sl] = gu * fv
                wc_v[sl] = fu * gv
                wd_v[sl] = fu * fv
                return c2

            lax.fori_loop(0, NG, build, 0, unroll=False)

            cp0 = pltpu.async_copy(t0.at[idx_v], g0_v, sem)
            cp1 = pltpu.async_copy(t1.at[idx_v], g1_v, sem)
            cp0.wait()
            cp1.wait()

            def blend(g, c2):
                sl = pl.ds(g * L, L)
                row = g * L + lanes
                wa = wa_v[sl]
                wb = wb_v[sl]
                wc = wc_v[sl]
                wd = wd_v[sl]
                for col, g_v in ((col0, g0_v), (col1, g1_v)):
                    a = g_v[sl]
                    b = g_v[pl.ds(K + g * L, L)]
                    cc = g_v[pl.ds(2 * K + g * L, L)]
                    d = g_v[pl.ds(3 * K + g * L, L)]
                    o = a * wa + b * wb + cc * wc + d * wd
                    plsc.store_scatter(o_v, [row, col], o)
                return c2

            lax.fori_loop(0, NG, blend, 0, unroll=False)

            pltpu.sync_copy(o_v, out_hbm.at[pl.ds(base, K)])
            return carry

        lax.fori_loop(0, NCHUNK, chunk_body, 0, unroll=False)

    return grid_sample


def kernel(velocity, points, bounding_box, grid_size):
    B, _, G, _ = velocity.shape
    N = points.shape[1]
    t0 = velocity[:, 0, :, :].reshape(B * G * G)
    t1 = velocity[:, 1, :, :].reshape(B * G * G)
    sx = (G - 1) / (bounding_box[0, 1] - bounding_box[0, 0])
    sy = (G - 1) / (bounding_box[1, 1] - bounding_box[1, 0])
    u = ((points[:, :, 0] - bounding_box[0, 0]) * sx).reshape(-1)
    v = ((points[:, :, 1] - bounding_box[1, 0]) * sy).reshape(-1)
    out = _make_kernel(B, N, G)(t0, t1, u, v)
    return out.reshape(B, N, 2)


# speedup vs baseline: 1.0775x; 1.0775x over previous
import functools

import jax
import jax.numpy as jnp
from jax import lax
from jax.experimental import pallas as pl
from jax.experimental.pallas import tpu as pltpu
from jax.experimental.pallas import tpu_sc as plsc

NC, NS, L = 2, 16, 16
NW = NC * NS


@functools.lru_cache(maxsize=None)
def _make_kernel(B, N, G):
    P = B * N
    assert P % NW == 0
    PW = P // NW
    assert N % PW == 0
    K = 2000
    assert PW % K == 0 and K % L == 0 and K % 8 == 0
    NCHUNK = PW // K
    NG = K // L
    GG = G * G
    TILES_PER_BATCH = NW // B

    mesh = plsc.VectorSubcoreMesh(core_axis_name="c", subcore_axis_name="s")

    @functools.partial(
        pl.kernel,
        out_type=jax.ShapeDtypeStruct((P, 2), jnp.float32),
        mesh=mesh,
        compiler_params=pltpu.CompilerParams(use_tc_tiling_on_sc=False,
                                             needs_layout_passes=False),
        scratch_types=[
            pltpu.VMEM((2, K), jnp.float32),
            pltpu.VMEM((2, K), jnp.float32),
            pltpu.VMEM((2, 4 * K), jnp.int32),
            pltpu.VMEM((2, K), jnp.float32),
            pltpu.VMEM((2, K), jnp.float32),
            pltpu.VMEM((2, K), jnp.float32),
            pltpu.VMEM((2, K), jnp.float32),
            pltpu.VMEM((2, 4 * K), jnp.float32),
            pltpu.VMEM((2, 4 * K), jnp.float32),
            pltpu.VMEM((K, 2), jnp.float32),
            pltpu.SemaphoreType.DMA,
        ],
    )
    def grid_sample(t0, t1, u_hbm, v_hbm, out_hbm,
                    u_v, v_v, idx_v, wa_v, wb_v, wc_v, wd_v, g0_v, g1_v, o_v,
                    sem):
        cid = lax.axis_index("c")
        sid = lax.axis_index("s")
        wid = sid * NC + cid
        batch = wid // TILES_PER_BATCH
        boff = batch * GG
        lanes = lax.iota(jnp.int32, L)
        col0 = jnp.zeros((L,), jnp.int32)
        col1 = jnp.ones((L,), jnp.int32)
        tbase = wid * PW

        def load_uv(ci, s):
            base = tbase + ci * K
            pltpu.sync_copy(u_hbm.at[pl.ds(base, K)], u_v.at[s])
            pltpu.sync_copy(v_hbm.at[pl.ds(base, K)], v_v.at[s])

        def build(s):
            def body(g, c2):
                sl = pl.ds(g * L, L)
                u = u_v[s, sl]
                v = v_v[s, sl]
                ui = jnp.minimum(u.astype(jnp.int32), G - 2)
                vi = jnp.minimum(v.astype(jnp.int32), G - 2)
                fu = u - ui.astype(jnp.float32)
                fv = v - vi.astype(jnp.float32)
                gu = 1.0 - fu
                gv = 1.0 - fv
                ia = ui * G + vi + boff
                idx_v[s, sl] = ia
                idx_v[s, pl.ds(K + g * L, L)] = ia + 1
                idx_v[s, pl.ds(2 * K + g * L, L)] = ia + G
                idx_v[s, pl.ds(3 * K + g * L, L)] = ia + G + 1
                wa_v[s, sl] = gu * gv
                wb_v[s, sl] = gu * fv
                wc_v[s, sl] = fu * gv
                wd_v[s, sl] = fu * fv
                return c2

            lax.fori_loop(0, NG, body, 0, unroll=False)

        def issue_gather(s):
            pltpu.async_copy(t0.at[idx_v.at[s]], g0_v.at[s], sem)
            pltpu.async_copy(t1.at[idx_v.at[s]], g1_v.at[s], sem)

        def wait_gather(s):
            pltpu.make_async_copy(t0.at[idx_v.at[s]], g0_v.at[s], sem).wait()
            pltpu.make_async_copy(t1.at[idx_v.at[s]], g1_v.at[s], sem).wait()

        def blend_store(ci, s):
            def body(g, c2):
                sl = pl.ds(g * L, L)
                row = g * L + lanes
                wa = wa_v[s, sl]
                wb = wb_v[s, sl]
                wc = wc_v[s, sl]
                wd = wd_v[s, sl]
                for col, g_v in ((col0, g0_v), (col1, g1_v)):
                    a = g_v[s, sl]
                    b = g_v[s, pl.ds(K + g * L, L)]
                    cc = g_v[s, pl.ds(2 * K + g * L, L)]
                    d = g_v[s, pl.ds(3 * K + g * L, L)]
                    o = a * wa + b * wb + cc * wc + d * wd
                    plsc.store_scatter(o_v, [row, col], o)
                return c2

            lax.fori_loop(0, NG, body, 0, unroll=False)
            pltpu.sync_copy(o_v, out_hbm.at[pl.ds(tbase + ci * K, K)])

        load_uv(0, 0)
        build(0)
        issue_gather(0)
        load_uv(1, 1)

        def chunk_body(ci, carry):
            p = lax.rem(ci, 2)
            q = 1 - p

            @pl.when(ci < NCHUNK - 1)
            def _():
                build(q)

            wait_gather(p)

            @pl.when(ci < NCHUNK - 1)
            def _():
                issue_gather(q)

            blend_store(ci, p)

            @pl.when(ci < NCHUNK - 2)
            def _():
                load_uv(ci + 2, p)

            return carry

        lax.fori_loop(0, NCHUNK, chunk_body, 0, unroll=False)

    return grid_sample


def kernel(velocity, points, bounding_box, grid_size):
    B, _, G, _ = velocity.shape
    N = points.shape[1]
    t0 = velocity[:, 0, :, :].reshape(B * G * G)
    t1 = velocity[:, 1, :, :].reshape(B * G * G)
    sx = (G - 1) / (bounding_box[0, 1] - bounding_box[0, 0])
    sy = (G - 1) / (bounding_box[1, 1] - bounding_box[1, 0])
    u = ((points[:, :, 0] - bounding_box[0, 0]) * sx).reshape(-1)
    v = ((points[:, :, 1] - bounding_box[1, 0]) * sy).reshape(-1)
    out = _make_kernel(B, N, G)(t0, t1, u, v)
    return out.reshape(B, N, 2)


# speedup vs baseline: 1.0782x; 1.0006x over previous
import functools

import jax
import jax.numpy as jnp
from jax import lax
from jax.experimental import pallas as pl
from jax.experimental.pallas import tpu as pltpu
from jax.experimental.pallas import tpu_sc as plsc

NC, NS, L = 2, 16, 16
NW = NC * NS


@functools.lru_cache(maxsize=None)
def _make_kernel(B, N, G):
    P = B * N
    assert P % NW == 0
    PW = P // NW
    assert N % PW == 0
    K = 2000
    assert PW % K == 0 and K % L == 0 and K % 8 == 0
    NCHUNK = PW // K
    NG = K // L
    GG = G * G
    TILES_PER_BATCH = NW // B

    mesh = plsc.VectorSubcoreMesh(core_axis_name="c", subcore_axis_name="s")

    @functools.partial(
        pl.kernel,
        out_type=jax.ShapeDtypeStruct((P, 8), jnp.float32),
        mesh=mesh,
        compiler_params=pltpu.CompilerParams(use_tc_tiling_on_sc=False,
                                             needs_layout_passes=False),
        scratch_types=[
            pltpu.VMEM((2, K), jnp.float32),
            pltpu.VMEM((2, K), jnp.float32),
            pltpu.VMEM((2, 4 * K), jnp.int32),
            pltpu.VMEM((2, K), jnp.float32),
            pltpu.VMEM((2, K), jnp.float32),
            pltpu.VMEM((2, K), jnp.float32),
            pltpu.VMEM((2, K), jnp.float32),
            pltpu.VMEM((2, 4 * K), jnp.float32),
            pltpu.VMEM((2, 4 * K), jnp.float32),
            pltpu.VMEM((K, 8), jnp.float32),
            pltpu.SemaphoreType.DMA,
        ],
    )
    def grid_sample(t0, t1, u_hbm, v_hbm, out_hbm,
                    u_v, v_v, idx_v, wa_v, wb_v, wc_v, wd_v, g0_v, g1_v, o_v,
                    sem):
        cid = lax.axis_index("c")
        sid = lax.axis_index("s")
        wid = sid * NC + cid
        batch = wid // TILES_PER_BATCH
        boff = batch * GG
        lanes = lax.iota(jnp.int32, L)
        col0 = jnp.zeros((L,), jnp.int32)
        col1 = jnp.ones((L,), jnp.int32)
        tbase = wid * PW

        def load_uv(ci, s):
            base = tbase + ci * K
            pltpu.sync_copy(u_hbm.at[pl.ds(base, K)], u_v.at[s])
            pltpu.sync_copy(v_hbm.at[pl.ds(base, K)], v_v.at[s])

        def build(s):
            def body(g, c2):
                sl = pl.ds(g * L, L)
                u = u_v[s, sl]
                v = v_v[s, sl]
                ui = jnp.minimum(u.astype(jnp.int32), G - 2)
                vi = jnp.minimum(v.astype(jnp.int32), G - 2)
                fu = u - ui.astype(jnp.float32)
                fv = v - vi.astype(jnp.float32)
                gu = 1.0 - fu
                gv = 1.0 - fv
                ia = ui * G + vi + boff
                idx_v[s, sl] = ia
                idx_v[s, pl.ds(K + g * L, L)] = ia + 1
                idx_v[s, pl.ds(2 * K + g * L, L)] = ia + G
                idx_v[s, pl.ds(3 * K + g * L, L)] = ia + G + 1
                wa_v[s, sl] = gu * gv
                wb_v[s, sl] = gu * fv
                wc_v[s, sl] = fu * gv
                wd_v[s, sl] = fu * fv
                return c2

            lax.fori_loop(0, NG, body, 0, unroll=False)

        def issue_gather(s):
            pltpu.async_copy(t0.at[idx_v.at[s]], g0_v.at[s], sem)
            pltpu.async_copy(t1.at[idx_v.at[s]], g1_v.at[s], sem)

        def wait_gather(s):
            pltpu.make_async_copy(t0.at[idx_v.at[s]], g0_v.at[s], sem).wait()
            pltpu.make_async_copy(t1.at[idx_v.at[s]], g1_v.at[s], sem).wait()

        def blend_store(ci, s):
            def body(g, c2):
                sl = pl.ds(g * L, L)
                row = g * L + lanes
                wa = wa_v[s, sl]
                wb = wb_v[s, sl]
                wc = wc_v[s, sl]
                wd = wd_v[s, sl]
                for col, g_v in ((col0, g0_v), (col1, g1_v)):
                    a = g_v[s, sl]
                    b = g_v[s, pl.ds(K + g * L, L)]
                    cc = g_v[s, pl.ds(2 * K + g * L, L)]
                    d = g_v[s, pl.ds(3 * K + g * L, L)]
                    o = a * wa + b * wb + cc * wc + d * wd
                    plsc.store_scatter(o_v, [row, col], o)
                return c2

            lax.fori_loop(0, NG, body, 0, unroll=False)
            pltpu.sync_copy(o_v, out_hbm.at[pl.ds(tbase + ci * K, K)])

        load_uv(0, 0)
        build(0)
        issue_gather(0)
        load_uv(1, 1)

        def chunk_body(ci, carry):
            p = lax.rem(ci, 2)
            q = 1 - p

            @pl.when(ci < NCHUNK - 1)
            def _():
                build(q)

            wait_gather(p)

            @pl.when(ci < NCHUNK - 1)
            def _():
                issue_gather(q)

            blend_store(ci, p)

            @pl.when(ci < NCHUNK - 2)
            def _():
                load_uv(ci + 2, p)

            return carry

        lax.fori_loop(0, NCHUNK, chunk_body, 0, unroll=False)

    return grid_sample


def kernel(velocity, points, bounding_box, grid_size):
    B, _, G, _ = velocity.shape
    N = points.shape[1]
    t0 = velocity[:, 0, :, :].reshape(B * G * G)
    t1 = velocity[:, 1, :, :].reshape(B * G * G)
    sx = (G - 1) / (bounding_box[0, 1] - bounding_box[0, 0])
    sy = (G - 1) / (bounding_box[1, 1] - bounding_box[1, 0])
    u = ((points[:, :, 0] - bounding_box[0, 0]) * sx).reshape(-1)
    v = ((points[:, :, 1] - bounding_box[1, 0]) * sy).reshape(-1)
    out = _make_kernel(B, N, G)(t0, t1, u, v)
    return out[:, :2].reshape(B, N, 2)
